# baseline (speedup 1.0000x reference)
import dataclasses
import functools

import jax
import jax.numpy as jnp
from jax import lax
from jax.experimental import pallas as pl
from jax.experimental.pallas import tpu as pltpu
from jax.experimental.pallas import tpu_sc as plsc

B = 16384
F = 64
NC = 2
NS = 16
NW = NC * NS
BPW = B // NW
L = 16


def _sc_pairmf(user, item_i, item_j, embed_user, embed_item):
    mesh = plsc.VectorSubcoreMesh(core_axis_name="c", subcore_axis_name="s")
    cp = pltpu.CompilerParams(
        needs_layout_passes=False, use_tc_tiling_on_sc=False
    )
    out_type = (
        jax.ShapeDtypeStruct((B,), jnp.float32),
        jax.ShapeDtypeStruct((B,), jnp.float32),
    )

    @functools.partial(
        pl.kernel,
        out_type=out_type,
        mesh=mesh,
        compiler_params=cp,
        scratch_types=[
            pltpu.VMEM((BPW,), jnp.int32),
            pltpu.VMEM((BPW,), jnp.int32),
            pltpu.VMEM((BPW,), jnp.int32),
            pltpu.VMEM((BPW, F), jnp.float32),
            pltpu.VMEM((BPW, F), jnp.float32),
            pltpu.VMEM((BPW, F), jnp.float32),
            pltpu.VMEM((BPW,), jnp.float32),
            pltpu.VMEM((BPW,), jnp.float32),
            pltpu.SemaphoreType.DMA,
            pltpu.SemaphoreType.DMA,
            pltpu.SemaphoreType.DMA,
        ],
    )
    def k(user_hbm, ii_hbm, ij_hbm, eu_hbm, ei_hbm, oi_hbm, oj_hbm,
          uidx, iidx, jidx, urows, irows, jrows, oi_v, oj_v, su, si, sj):
        wid = lax.axis_index("s") * NC + lax.axis_index("c")
        base = wid * BPW

        pltpu.sync_copy(user_hbm.at[pl.ds(base, BPW)], uidx)
        pltpu.sync_copy(ii_hbm.at[pl.ds(base, BPW)], iidx)
        pltpu.sync_copy(ij_hbm.at[pl.ds(base, BPW)], jidx)

        cu = pltpu.async_copy(eu_hbm.at[uidx], urows, su)
        ci = pltpu.async_copy(ei_hbm.at[iidx], irows, si)
        cj = pltpu.async_copy(ei_hbm.at[jidx], jrows, sj)
        cu.wait()
        ci.wait()
        cj.wait()

        lane = lax.iota(jnp.int32, L)
        m15 = lane == (L - 1)

        @pl.loop(0, BPW)
        def _(r):
            u0 = urows[r, pl.ds(0, L)]
            u1 = urows[r, pl.ds(L, L)]
            u2 = urows[r, pl.ds(2 * L, L)]
            u3 = urows[r, pl.ds(3 * L, L)]
            a0 = irows[r, pl.ds(0, L)]
            a1 = irows[r, pl.ds(L, L)]
            a2 = irows[r, pl.ds(2 * L, L)]
            a3 = irows[r, pl.ds(3 * L, L)]
            b0 = jrows[r, pl.ds(0, L)]
            b1 = jrows[r, pl.ds(L, L)]
            b2 = jrows[r, pl.ds(2 * L, L)]
            b3 = jrows[r, pl.ds(3 * L, L)]
            acc_i = u0 * a0 + u1 * a1 + u2 * a2 + u3 * a3
            acc_j = u0 * b0 + u1 * b1 + u2 * b2 + u3 * b3
            ridx = jnp.full((L,), r, jnp.int32)
            plsc.store_scatter(oi_v, [ridx], plsc.cumsum(acc_i), mask=m15)
            plsc.store_scatter(oj_v, [ridx], plsc.cumsum(acc_j), mask=m15)

        pltpu.sync_copy(oi_v, oi_hbm.at[pl.ds(base, BPW)])
        pltpu.sync_copy(oj_v, oj_hbm.at[pl.ds(base, BPW)])

    return k(user, item_i, item_j, embed_user, embed_item)


def kernel(user, item_i, item_j, embed_user, embed_item):
    user = user.astype(jnp.int32)
    item_i = item_i.astype(jnp.int32)
    item_j = item_j.astype(jnp.int32)
    return _sc_pairmf(user, item_i, item_j, embed_user, embed_item)
